# Initial kernel scaffold; baseline (speedup 1.0000x reference)
import jax
import jax.numpy as jnp
from jax.experimental import pallas as pl


def kernel(x, edge_index, W, a):
    raise NotImplementedError("write your pallas kernel here")


# speedup vs baseline: 21.4242x; 21.4242x over previous
import dataclasses

import jax
import jax.numpy as jnp
from jax import lax
from jax.experimental import pallas as pl
from jax.experimental.pallas import tpu as pltpu
from jax.experimental.pallas import tpu_sc as plsc

N = 10000
E = 320000
D = 128
ALPHA = 0.2
NC = 2
NS = 16
NW = NC * NS
EPW = E // NW
CH = 80
NCH = EPW // CH
LANES = 16


def _sc_compiler_params():
    cp = pltpu.CompilerParams()
    fields = pltpu.CompilerParams.__dataclass_fields__
    if "needs_layout_passes" in fields:
        cp = dataclasses.replace(cp, needs_layout_passes=False)
    if "use_tc_tiling_on_sc" in fields:
        cp = dataclasses.replace(cp, use_tc_tiling_on_sc=False)
    return cp


def _prep_body(x_ref, w_ref, aa_ref, h_ref, s_ref, c_ref):
    h = jnp.dot(x_ref[...], w_ref[...], preferred_element_type=jnp.float32)
    h_ref[...] = h
    s = jnp.dot(h, aa_ref[...], preferred_element_type=jnp.float32)
    s_ref[...] = s
    c = jnp.maximum(jnp.max(s[:, 0]) + jnp.max(s[:, 1]), 0.0)
    c_ref[...] = jnp.broadcast_to(c, (LANES,))


def _finish_body(ssum_ref, p_ref, o_ref):
    ssum = jnp.sum(ssum_ref[...], axis=0)
    sinv = 1.0 / jnp.maximum(ssum, 1e-8)
    p = p_ref[...]
    acc = (p[0] + p[1]) * sinv[:, None]
    o_ref[...] = jnp.where(acc > 0, acc, jnp.exp(acc) - 1.0)


def _edge_w_body(row_hbm, col_hbm, s1_hbm, s2_hbm, c_hbm,
                 w_out_hbm, ssum_hbm,
                 s1_ts, s2_ts, row_ts, col_ts, w_ts, ssum_ts, c_ts):
    cid = lax.axis_index("c")
    sid = lax.axis_index("s")
    wid = sid * NC + cid

    pltpu.sync_copy(s1_hbm, s1_ts)
    pltpu.sync_copy(s2_hbm, s2_ts)
    pltpu.sync_copy(row_hbm.at[wid], row_ts)
    pltpu.sync_copy(col_hbm.at[wid], col_ts)
    pltpu.sync_copy(c_hbm, c_ts)
    cvec = c_ts[...]

    zero = jnp.zeros((LANES,), jnp.float32)

    @pl.loop(0, N, step=LANES)
    def _(i):
        ssum_ts[pl.ds(i, LANES)] = zero

    @pl.loop(0, NCH)
    def _(j):
        @pl.loop(0, CH, step=LANES)
        def _(k):
            r = row_ts[j, pl.ds(k, LANES)]
            c = col_ts[j, pl.ds(k, LANES)]
            g1 = plsc.load_gather(s1_ts, [r])
            g2 = plsc.load_gather(s2_ts, [c])
            z = g1 + g2
            e = jnp.maximum(z, ALPHA * z)
            w = jnp.exp(e - cvec)
            w_ts[j, pl.ds(k, LANES)] = w
            plsc.addupdate_scatter(ssum_ts, [r], w)

    pltpu.sync_copy(w_ts, w_out_hbm.at[wid])
    pltpu.sync_copy(ssum_ts, ssum_hbm.at[wid])


def _spmm_body(row_hbm, col_hbm, w_hbm, h_hbm, outp_hbm,
               row_ts, col_ts, w_ts, hbuf, out_sh):
    cid = lax.axis_index("c")
    sid = lax.axis_index("s")
    wid = sid * NC + cid

    pltpu.sync_copy(row_hbm.at[wid], row_ts)
    pltpu.sync_copy(col_hbm.at[wid], col_ts)
    pltpu.sync_copy(w_hbm.at[wid], w_ts)

    zero = jnp.zeros((LANES,), jnp.float32)

    @pl.loop(0, CH)
    def _(i):
        for t in range(D // LANES):
            hbuf[i, pl.ds(t * LANES, LANES)] = zero

    nrowch = N // CH
    for i in range((nrowch + NS - 1) // NS):
        c = i * NS + sid

        @pl.when(c < nrowch)
        def _():
            pltpu.sync_copy(hbuf, out_sh.at[pl.ds(c * CH, CH)])

    plsc.subcore_barrier()

    @pl.loop(0, NCH)
    def _(j):
        pltpu.sync_copy(h_hbm.at[col_ts.at[j]], hbuf)

        @pl.loop(0, CH)
        def _(k):
            jv = jnp.full((LANES,), j, jnp.int32)
            kv = jnp.full((LANES,), k, jnp.int32)
            wv = plsc.load_gather(w_ts, [jv, kv])
            for t in range(D // LANES):
                sl = pl.ds(t * LANES, LANES)
                hbuf[k, sl] = hbuf[k, sl] * wv

        pltpu.sync_copy(hbuf, out_sh.at[row_ts.at[j]], add=True)

    plsc.subcore_barrier()
    for i in range((nrowch + NS - 1) // NS):
        c = i * NS + sid

        @pl.when(c < nrowch)
        def _():
            pltpu.sync_copy(out_sh.at[pl.ds(c * CH, CH)],
                            outp_hbm.at[cid, pl.ds(c * CH, CH)])


def kernel(x, edge_index, W, a):
    aa = a.reshape(2, D).T
    row3 = edge_index[0].reshape(NW, NCH, CH)
    col3 = edge_index[1].reshape(NW, NCH, CH)

    h, s, cvec = pl.pallas_call(
        _prep_body,
        out_shape=[
            jax.ShapeDtypeStruct((N, D), jnp.float32),
            jax.ShapeDtypeStruct((N, 2), jnp.float32),
            jax.ShapeDtypeStruct((LANES,), jnp.float32),
        ],
    )(x, W, aa)
    s1 = s[:, 0]
    s2 = s[:, 1]

    cp = _sc_compiler_params()
    mesh = plsc.VectorSubcoreMesh(core_axis_name="c", subcore_axis_name="s")

    edge_w_fn = pl.kernel(
        _edge_w_body,
        out_type=[
            jax.ShapeDtypeStruct((NW, NCH, CH), jnp.float32),
            jax.ShapeDtypeStruct((NW, N), jnp.float32),
        ],
        mesh=mesh,
        scratch_types=[
            pltpu.VMEM((N,), jnp.float32),
            pltpu.VMEM((N,), jnp.float32),
            pltpu.VMEM((NCH, CH), jnp.int32),
            pltpu.VMEM((NCH, CH), jnp.int32),
            pltpu.VMEM((NCH, CH), jnp.float32),
            pltpu.VMEM((N,), jnp.float32),
            pltpu.VMEM((LANES,), jnp.float32),
        ],
        compiler_params=cp,
    )
    w3, ssum_p = edge_w_fn(row3, col3, s1, s2, cvec)

    spmm_fn = pl.kernel(
        _spmm_body,
        out_type=jax.ShapeDtypeStruct((NC, N, D), jnp.float32),
        mesh=mesh,
        scratch_types=[
            pltpu.VMEM((NCH, CH), jnp.int32),
            pltpu.VMEM((NCH, CH), jnp.int32),
            pltpu.VMEM((NCH, CH), jnp.float32),
            pltpu.VMEM((CH, D), jnp.float32),
            pltpu.VMEM_SHARED((N, D), jnp.float32),
        ],
        compiler_params=cp,
    )
    outp = spmm_fn(row3, col3, w3, h)

    out = pl.pallas_call(
        _finish_body,
        out_shape=jax.ShapeDtypeStruct((N, D), jnp.float32),
    )(ssum_p, outp)
    return out


# speedup vs baseline: 31.9186x; 1.4898x over previous
import dataclasses

import jax
import jax.numpy as jnp
from jax import lax
from jax.experimental import pallas as pl
from jax.experimental.pallas import tpu as pltpu
from jax.experimental.pallas import tpu_sc as plsc

N = 10000
E = 320000
D = 128
ALPHA = 0.2
NC = 2
NS = 16
NW = NC * NS
EPW = E // NW
CH = 80
NCH = EPW // CH
LANES = 16


def _sc_compiler_params():
    cp = pltpu.CompilerParams()
    fields = pltpu.CompilerParams.__dataclass_fields__
    if "needs_layout_passes" in fields:
        cp = dataclasses.replace(cp, needs_layout_passes=False)
    if "use_tc_tiling_on_sc" in fields:
        cp = dataclasses.replace(cp, use_tc_tiling_on_sc=False)
    return cp


def _prep_body(x_ref, w_ref, aa_ref, h_ref, s_ref, c_ref):
    h = jnp.dot(x_ref[...], w_ref[...], preferred_element_type=jnp.float32)
    h_ref[...] = h
    s = jnp.dot(h, aa_ref[...], preferred_element_type=jnp.float32)
    s_ref[...] = s
    c = jnp.maximum(jnp.max(s[:, 0]) + jnp.max(s[:, 1]), 0.0)
    c_ref[...] = jnp.broadcast_to(c, (LANES,))


def _finish_body(ssum_ref, p_ref, o_ref):
    ssum = jnp.sum(ssum_ref[...], axis=0)
    sinv = 1.0 / jnp.maximum(ssum, 1e-8)
    p = p_ref[...]
    acc = (p[0] + p[1]) * sinv[:, None]
    o_ref[...] = jnp.where(acc > 0, acc, jnp.exp(acc) - 1.0)


def _edge_w_body(row_hbm, col_hbm, s1_hbm, s2_hbm, c_hbm,
                 w_out_hbm, ssum_hbm,
                 s1_ts, s2_ts, row_ts, col_ts, w_ts, ssum_ts, c_ts):
    cid = lax.axis_index("c")
    sid = lax.axis_index("s")
    wid = sid * NC + cid

    pltpu.sync_copy(s1_hbm, s1_ts)
    pltpu.sync_copy(s2_hbm, s2_ts)
    pltpu.sync_copy(row_hbm.at[wid], row_ts)
    pltpu.sync_copy(col_hbm.at[wid], col_ts)
    pltpu.sync_copy(c_hbm, c_ts)
    cvec = c_ts[...]

    zero = jnp.zeros((LANES,), jnp.float32)

    @pl.loop(0, N, step=LANES)
    def _(i):
        ssum_ts[pl.ds(i, LANES)] = zero

    @pl.loop(0, NCH)
    def _(j):
        @pl.loop(0, CH, step=LANES)
        def _(k):
            r = row_ts[j, pl.ds(k, LANES)]
            c = col_ts[j, pl.ds(k, LANES)]
            g1 = plsc.load_gather(s1_ts, [r])
            g2 = plsc.load_gather(s2_ts, [c])
            z = g1 + g2
            e = jnp.maximum(z, ALPHA * z)
            w = jnp.exp(e - cvec)
            w_ts[pl.ds(j * CH + k, LANES)] = w
            plsc.addupdate_scatter(ssum_ts, [r], w)

    pltpu.sync_copy(w_ts, w_out_hbm.at[wid])
    pltpu.sync_copy(ssum_ts, ssum_hbm.at[wid])


def _spmm_body(row_hbm, col_hbm, w_hbm, h_hbm, outp_hbm,
               row_ts, col_ts, w_ts, hbuf0, hbuf1, out_sh,
               g0sem, g1sem, s0sem, s1sem):
    cid = lax.axis_index("c")
    sid = lax.axis_index("s")
    wid = sid * NC + cid

    pltpu.sync_copy(row_hbm.at[wid], row_ts)
    pltpu.sync_copy(col_hbm.at[wid], col_ts)
    pltpu.sync_copy(w_hbm.at[wid], w_ts)

    zero = jnp.zeros((LANES,), jnp.float32)

    @pl.loop(0, CH)
    def _(i):
        for t in range(D // LANES):
            hbuf0[i, pl.ds(t * LANES, LANES)] = zero

    nrowch = N // CH
    for i in range((nrowch + NS - 1) // NS):
        c = i * NS + sid

        @pl.when(c < nrowch)
        def _():
            pltpu.sync_copy(hbuf0, out_sh.at[pl.ds(c * CH, CH)])

    plsc.subcore_barrier()

    def scale(buf, j):
        @pl.loop(0, CH, step=4)
        def _(kk):
            for u in range(4):
                k = kk + u
                wv = plsc.load_gather(
                    w_ts, [jnp.full((LANES,), j * CH + k, jnp.int32)])
                for t in range(D // LANES):
                    sl = pl.ds(t * LANES, LANES)
                    buf[k, sl] = buf[k, sl] * wv

    def gather_start(buf, j, sem):
        pltpu.async_copy(h_hbm.at[col_ts.at[j]], buf, sem)

    def gather_wait(buf, j, sem):
        pltpu.make_async_copy(h_hbm.at[col_ts.at[j]], buf, sem).wait()

    def scatter_start(buf, j, sem):
        pltpu.async_copy(buf, out_sh.at[row_ts.at[j]], sem, add=True)

    def scatter_wait(buf, j, sem):
        pltpu.make_async_copy(buf, out_sh.at[row_ts.at[j]], sem).wait()

    gather_start(hbuf0, 0, g0sem)

    @pl.loop(0, (NCH - 1) // 2)
    def _(m):
        j0 = 2 * m
        j1 = j0 + 1

        @pl.when(m > 0)
        def _():
            scatter_wait(hbuf1, j1 - 2, s1sem)

        gather_start(hbuf1, j1, g1sem)
        gather_wait(hbuf0, j0, g0sem)
        scale(hbuf0, j0)
        scatter_start(hbuf0, j0, s0sem)
        gather_wait(hbuf1, j1, g1sem)
        scale(hbuf1, j1)
        scatter_start(hbuf1, j1, s1sem)
        scatter_wait(hbuf0, j0, s0sem)
        gather_start(hbuf0, j0 + 2, g0sem)

    last = NCH - 1
    gather_wait(hbuf0, last, g0sem)
    scale(hbuf0, last)
    scatter_start(hbuf0, last, s0sem)
    scatter_wait(hbuf0, last, s0sem)
    scatter_wait(hbuf1, last - 1, s1sem)

    plsc.subcore_barrier()
    for i in range((nrowch + NS - 1) // NS):
        c = i * NS + sid

        @pl.when(c < nrowch)
        def _():
            pltpu.sync_copy(out_sh.at[pl.ds(c * CH, CH)],
                            outp_hbm.at[cid, pl.ds(c * CH, CH)])


def kernel(x, edge_index, W, a):
    aa = a.reshape(2, D).T
    row3 = edge_index[0].reshape(NW, NCH, CH)
    col3 = edge_index[1].reshape(NW, NCH, CH)

    h, s, cvec = pl.pallas_call(
        _prep_body,
        out_shape=[
            jax.ShapeDtypeStruct((N, D), jnp.float32),
            jax.ShapeDtypeStruct((N, 2), jnp.float32),
            jax.ShapeDtypeStruct((LANES,), jnp.float32),
        ],
    )(x, W, aa)
    s1 = s[:, 0]
    s2 = s[:, 1]

    cp = _sc_compiler_params()
    mesh = plsc.VectorSubcoreMesh(core_axis_name="c", subcore_axis_name="s")

    edge_w_fn = pl.kernel(
        _edge_w_body,
        out_type=[
            jax.ShapeDtypeStruct((NW, EPW), jnp.float32),
            jax.ShapeDtypeStruct((NW, N), jnp.float32),
        ],
        mesh=mesh,
        scratch_types=[
            pltpu.VMEM((N,), jnp.float32),
            pltpu.VMEM((N,), jnp.float32),
            pltpu.VMEM((NCH, CH), jnp.int32),
            pltpu.VMEM((NCH, CH), jnp.int32),
            pltpu.VMEM((EPW,), jnp.float32),
            pltpu.VMEM((N,), jnp.float32),
            pltpu.VMEM((LANES,), jnp.float32),
        ],
        compiler_params=cp,
    )
    w3, ssum_p = edge_w_fn(row3, col3, s1, s2, cvec)

    spmm_fn = pl.kernel(
        _spmm_body,
        out_type=jax.ShapeDtypeStruct((NC, N, D), jnp.float32),
        mesh=mesh,
        scratch_types=[
            pltpu.VMEM((NCH, CH), jnp.int32),
            pltpu.VMEM((NCH, CH), jnp.int32),
            pltpu.VMEM((EPW,), jnp.float32),
            pltpu.VMEM((CH, D), jnp.float32),
            pltpu.VMEM((CH, D), jnp.float32),
            pltpu.VMEM_SHARED((N, D), jnp.float32),
            pltpu.SemaphoreType.DMA,
            pltpu.SemaphoreType.DMA,
            pltpu.SemaphoreType.DMA,
            pltpu.SemaphoreType.DMA,
        ],
        compiler_params=cp,
    )
    outp = spmm_fn(row3, col3, w3, h)

    out = pl.pallas_call(
        _finish_body,
        out_shape=jax.ShapeDtypeStruct((N, D), jnp.float32),
    )(ssum_p, outp)
    return out


# speedup vs baseline: 32.1090x; 1.0060x over previous
import dataclasses

import jax
import jax.numpy as jnp
import numpy as np
from jax import lax
from jax.experimental import pallas as pl
from jax.experimental.pallas import tpu as pltpu
from jax.experimental.pallas import tpu_sc as plsc

N = 10000
E = 320000
D = 128
ALPHA = 0.2
NC = 2
NS = 16
NW = NC * NS
EPW = E // NW
CH = 80
NCH = EPW // CH
LANES = 16


def _sc_compiler_params():
    cp = pltpu.CompilerParams()
    fields = pltpu.CompilerParams.__dataclass_fields__
    if "needs_layout_passes" in fields:
        cp = dataclasses.replace(cp, needs_layout_passes=False)
    if "use_tc_tiling_on_sc" in fields:
        cp = dataclasses.replace(cp, use_tc_tiling_on_sc=False)
    return cp


def _prep_body(x_ref, w_ref, aa_ref, h_ref, s_ref, c_ref):
    h = jnp.dot(x_ref[...], w_ref[...], preferred_element_type=jnp.float32)
    h_ref[...] = h
    s = jnp.dot(h, aa_ref[...], preferred_element_type=jnp.float32)
    s_ref[...] = s
    c = jnp.maximum(jnp.max(s[:, 0]) + jnp.max(s[:, 1]), 0.0)
    c_ref[...] = jnp.broadcast_to(c, (LANES,))


def _finish_body(ssum_ref, p_ref, o_ref):
    ssum = jnp.sum(ssum_ref[...], axis=0)
    sinv = 1.0 / jnp.maximum(ssum, 1e-8)
    p = p_ref[...]
    acc = (p[0] + p[1]) * sinv[:, None]
    o_ref[...] = jnp.where(acc > 0, acc, jnp.exp(acc) - 1.0)


def _edge_w_body(row_hbm, col_hbm, s1_hbm, s2_hbm, c_hbm,
                 w_out_hbm, ssum_hbm,
                 s1_ts, s2_ts, row_ts, col_ts, w_ts, ssum_ts, c_ts):
    cid = lax.axis_index("c")
    sid = lax.axis_index("s")
    wid = sid * NC + cid

    pltpu.sync_copy(s1_hbm, s1_ts)
    pltpu.sync_copy(s2_hbm, s2_ts)
    pltpu.sync_copy(row_hbm.at[wid], row_ts)
    pltpu.sync_copy(col_hbm.at[wid], col_ts)
    pltpu.sync_copy(c_hbm, c_ts)
    cvec = c_ts[...]

    zero = jnp.zeros((LANES,), jnp.float32)

    @pl.loop(0, N, step=LANES)
    def _(i):
        ssum_ts[pl.ds(i, LANES)] = zero

    @pl.loop(0, NCH)
    def _(j):
        @pl.loop(0, CH, step=LANES)
        def _(k):
            r = row_ts[j, pl.ds(k, LANES)]
            c = col_ts[j, pl.ds(k, LANES)]
            g1 = plsc.load_gather(s1_ts, [r])
            g2 = plsc.load_gather(s2_ts, [c])
            z = g1 + g2
            e = jnp.maximum(z, ALPHA * z)
            w = jnp.exp(e - cvec)
            w_ts[pl.ds(j * CH + k, LANES)] = w
            plsc.addupdate_scatter(ssum_ts, [r], w)

    pltpu.sync_copy(w_ts, w_out_hbm.at[wid])
    pltpu.sync_copy(ssum_ts, ssum_hbm.at[wid])


def _spmm_body(row_hbm, col_hbm, w_hbm, h_hbm, outp_hbm,
               row_ts, col_ts, w_ts, hbuf0, hbuf1, out_sh,
               g0sem, g1sem, s0sem, s1sem):
    cid = lax.axis_index("c")
    sid = lax.axis_index("s")
    wid = sid * NC + cid

    pltpu.sync_copy(row_hbm.at[wid], row_ts)
    pltpu.sync_copy(col_hbm.at[wid], col_ts)
    pltpu.sync_copy(w_hbm.at[wid], w_ts)

    zero = jnp.zeros((LANES,), jnp.float32)

    @pl.loop(0, CH)
    def _(i):
        for t in range(D // LANES):
            hbuf0[i, pl.ds(t * LANES, LANES)] = zero

    nrowch = N // CH
    for i in range((nrowch + NS - 1) // NS):
        c = i * NS + sid

        @pl.when(c < nrowch)
        def _():
            pltpu.sync_copy(hbuf0, out_sh.at[pl.ds(c * CH, CH)])

    plsc.subcore_barrier()

    def scale(buf, j):
        @pl.loop(0, CH, step=LANES)
        def _(kk):
            base = j * CH + kk
            for u in range(LANES):
                wv = plsc.load_gather(
                    w_ts, [jnp.full((LANES,), base + u, jnp.int32)])
                for t in range(D // LANES):
                    sl = pl.ds(t * LANES, LANES)
                    buf[kk + u, sl] = buf[kk + u, sl] * wv

    def gather_start(buf, j, sem):
        pltpu.async_copy(h_hbm.at[col_ts.at[j]], buf, sem)

    def gather_wait(buf, j, sem):
        pltpu.make_async_copy(h_hbm.at[col_ts.at[j]], buf, sem).wait()

    def scatter_start(buf, j, sem):
        pltpu.async_copy(buf, out_sh.at[row_ts.at[j]], sem, add=True)

    def scatter_wait(buf, j, sem):
        pltpu.make_async_copy(buf, out_sh.at[row_ts.at[j]], sem).wait()

    gather_start(hbuf0, 0, g0sem)

    @pl.loop(0, (NCH - 1) // 2)
    def _(m):
        j0 = 2 * m
        j1 = j0 + 1

        @pl.when(m > 0)
        def _():
            scatter_wait(hbuf1, j1 - 2, s1sem)

        gather_start(hbuf1, j1, g1sem)
        gather_wait(hbuf0, j0, g0sem)
        scale(hbuf0, j0)
        scatter_start(hbuf0, j0, s0sem)
        gather_wait(hbuf1, j1, g1sem)
        scale(hbuf1, j1)
        scatter_start(hbuf1, j1, s1sem)
        scatter_wait(hbuf0, j0, s0sem)
        gather_start(hbuf0, j0 + 2, g0sem)

    last = NCH - 1
    gather_wait(hbuf0, last, g0sem)
    scale(hbuf0, last)
    scatter_start(hbuf0, last, s0sem)
    scatter_wait(hbuf0, last, s0sem)
    scatter_wait(hbuf1, last - 1, s1sem)

    plsc.subcore_barrier()
    for i in range((nrowch + NS - 1) // NS):
        c = i * NS + sid

        @pl.when(c < nrowch)
        def _():
            pltpu.sync_copy(out_sh.at[pl.ds(c * CH, CH)],
                            outp_hbm.at[cid, pl.ds(c * CH, CH)])


def kernel(x, edge_index, W, a):
    aa = a.reshape(2, D).T
    row3 = edge_index[0].reshape(NW, NCH, CH)
    col3 = edge_index[1].reshape(NW, NCH, CH)

    h, s, cvec = pl.pallas_call(
        _prep_body,
        out_shape=[
            jax.ShapeDtypeStruct((N, D), jnp.float32),
            jax.ShapeDtypeStruct((N, 2), jnp.float32),
            jax.ShapeDtypeStruct((LANES,), jnp.float32),
        ],
    )(x, W, aa)
    s1 = s[:, 0]
    s2 = s[:, 1]

    cp = _sc_compiler_params()
    mesh = plsc.VectorSubcoreMesh(core_axis_name="c", subcore_axis_name="s")

    edge_w_fn = pl.kernel(
        _edge_w_body,
        out_type=[
            jax.ShapeDtypeStruct((NW, EPW), jnp.float32),
            jax.ShapeDtypeStruct((NW, N), jnp.float32),
        ],
        mesh=mesh,
        scratch_types=[
            pltpu.VMEM((N,), jnp.float32),
            pltpu.VMEM((N,), jnp.float32),
            pltpu.VMEM((NCH, CH), jnp.int32),
            pltpu.VMEM((NCH, CH), jnp.int32),
            pltpu.VMEM((EPW,), jnp.float32),
            pltpu.VMEM((N,), jnp.float32),
            pltpu.VMEM((LANES,), jnp.float32),
        ],
        compiler_params=cp,
    )
    w3, ssum_p = edge_w_fn(row3, col3, s1, s2, cvec)

    spmm_fn = pl.kernel(
        _spmm_body,
        out_type=jax.ShapeDtypeStruct((NC, N, D), jnp.float32),
        mesh=mesh,
        scratch_types=[
            pltpu.VMEM((NCH, CH), jnp.int32),
            pltpu.VMEM((NCH, CH), jnp.int32),
            pltpu.VMEM((EPW,), jnp.float32),
            pltpu.VMEM((CH, D), jnp.float32),
            pltpu.VMEM((CH, D), jnp.float32),
            pltpu.VMEM_SHARED((N, D), jnp.float32),
            pltpu.SemaphoreType.DMA,
            pltpu.SemaphoreType.DMA,
            pltpu.SemaphoreType.DMA,
            pltpu.SemaphoreType.DMA,
        ],
        compiler_params=cp,
    )
    outp = spmm_fn(row3, col3, w3, h)

    out = pl.pallas_call(
        _finish_body,
        out_shape=jax.ShapeDtypeStruct((N, D), jnp.float32),
    )(ssum_p, outp)
    return out


# speedup vs baseline: 33.7336x; 1.0506x over previous
import dataclasses

import jax
import jax.numpy as jnp
import numpy as np
from jax import lax
from jax.experimental import pallas as pl
from jax.experimental.pallas import tpu as pltpu
from jax.experimental.pallas import tpu_sc as plsc

N = 10000
E = 320000
D = 128
ALPHA = 0.2
NC = 2
NS = 16
NW = NC * NS
EPW = E // NW
CH = 80
NCH = EPW // CH
LANES = 16


def _sc_compiler_params():
    cp = pltpu.CompilerParams()
    fields = pltpu.CompilerParams.__dataclass_fields__
    if "needs_layout_passes" in fields:
        cp = dataclasses.replace(cp, needs_layout_passes=False)
    if "use_tc_tiling_on_sc" in fields:
        cp = dataclasses.replace(cp, use_tc_tiling_on_sc=False)
    return cp


def _prep_body(x_ref, w_ref, aa_ref, hb_ref, s_ref, c_ref):
    h = jnp.dot(x_ref[...], w_ref[...], preferred_element_type=jnp.float32)
    hb_ref[...] = h.astype(jnp.bfloat16)
    s = jnp.dot(h, aa_ref[...], preferred_element_type=jnp.float32)
    s_ref[...] = s
    c = jnp.maximum(jnp.max(s[:, 0]) + jnp.max(s[:, 1]), 0.0)
    c_ref[...] = jnp.broadcast_to(c, (LANES,))


def _finish_body(ssum_ref, p_ref, o_ref):
    ssum = jnp.sum(ssum_ref[...], axis=0)
    sinv = 1.0 / jnp.maximum(ssum, 1e-8)
    p = p_ref[...].astype(jnp.float32)
    acc = (p[0] + p[1]) * sinv[:, None]
    o_ref[...] = jnp.where(acc > 0, acc, jnp.exp(acc) - 1.0)


def _edge_w_body(row_hbm, col_hbm, s1_hbm, s2_hbm, c_hbm,
                 w_out_hbm, ssum_hbm,
                 s1_ts, s2_ts, row_ts, col_ts, w_ts, ssum_ts, c_ts):
    cid = lax.axis_index("c")
    sid = lax.axis_index("s")
    wid = sid * NC + cid

    pltpu.sync_copy(s1_hbm, s1_ts)
    pltpu.sync_copy(s2_hbm, s2_ts)
    pltpu.sync_copy(row_hbm.at[wid], row_ts)
    pltpu.sync_copy(col_hbm.at[wid], col_ts)
    pltpu.sync_copy(c_hbm, c_ts)
    cvec = c_ts[...]

    zero = jnp.zeros((LANES,), jnp.float32)

    @pl.loop(0, N, step=LANES)
    def _(i):
        ssum_ts[pl.ds(i, LANES)] = zero

    @pl.loop(0, NCH)
    def _(j):
        @pl.loop(0, CH, step=LANES)
        def _(k):
            r = row_ts[j, pl.ds(k, LANES)]
            c = col_ts[j, pl.ds(k, LANES)]
            g1 = plsc.load_gather(s1_ts, [r])
            g2 = plsc.load_gather(s2_ts, [c])
            z = g1 + g2
            e = jnp.maximum(z, ALPHA * z)
            w = jnp.exp(e - cvec)
            w_ts[pl.ds(j * CH + k, LANES)] = w
            plsc.addupdate_scatter(ssum_ts, [r], w)

    pltpu.sync_copy(w_ts, w_out_hbm.at[wid])
    pltpu.sync_copy(ssum_ts, ssum_hbm.at[wid])


def _spmm_body(row_hbm, col_hbm, w_hbm, h_hbm, outp_hbm,
               row_ts, col_ts, w_ts, hbuf0, hbuf1, out_sh,
               g0sem, g1sem, s0sem, s1sem):
    cid = lax.axis_index("c")
    sid = lax.axis_index("s")
    wid = sid * NC + cid

    pltpu.sync_copy(row_hbm.at[wid], row_ts)
    pltpu.sync_copy(col_hbm.at[wid], col_ts)
    pltpu.sync_copy(w_hbm.at[wid], w_ts)

    zero = jnp.zeros((2 * LANES,), jnp.bfloat16)

    @pl.loop(0, CH)
    def _(i):
        for t in range(D // (2 * LANES)):
            hbuf0[i, pl.ds(t * 2 * LANES, 2 * LANES)] = zero

    nrowch = N // CH
    for i in range((nrowch + NS - 1) // NS):
        c = i * NS + sid

        @pl.when(c < nrowch)
        def _():
            pltpu.sync_copy(hbuf0, out_sh.at[pl.ds(c * CH, CH)])

    plsc.subcore_barrier()

    def scale(buf, j):
        @pl.loop(0, CH, step=LANES)
        def _(kk):
            base = j * CH + kk
            for u in range(LANES):
                wv = plsc.load_gather(
                    w_ts, [jnp.full((LANES,), base + u, jnp.int32)])
                wb = plsc.pack(wv, wv, format=plsc.PackFormat.INTERLEAVED)
                for t in range(D // (2 * LANES)):
                    sl = pl.ds(t * 2 * LANES, 2 * LANES)
                    buf[kk + u, sl] = buf[kk + u, sl] * wb

    def gather_start(buf, j, sem):
        pltpu.async_copy(h_hbm.at[col_ts.at[j]], buf, sem)

    def gather_wait(buf, j, sem):
        pltpu.make_async_copy(h_hbm.at[col_ts.at[j]], buf, sem).wait()

    def scatter_start(buf, j, sem):
        pltpu.async_copy(buf, out_sh.at[row_ts.at[j]], sem, add=True)

    def scatter_wait(buf, j, sem):
        pltpu.make_async_copy(buf, out_sh.at[row_ts.at[j]], sem).wait()

    gather_start(hbuf0, 0, g0sem)

    @pl.loop(0, (NCH - 1) // 2)
    def _(m):
        j0 = 2 * m
        j1 = j0 + 1

        @pl.when(m > 0)
        def _():
            scatter_wait(hbuf1, j1 - 2, s1sem)

        gather_start(hbuf1, j1, g1sem)
        gather_wait(hbuf0, j0, g0sem)
        scale(hbuf0, j0)
        scatter_start(hbuf0, j0, s0sem)
        gather_wait(hbuf1, j1, g1sem)
        scale(hbuf1, j1)
        scatter_start(hbuf1, j1, s1sem)
        scatter_wait(hbuf0, j0, s0sem)
        gather_start(hbuf0, j0 + 2, g0sem)

    last = NCH - 1
    gather_wait(hbuf0, last, g0sem)
    scale(hbuf0, last)
    scatter_start(hbuf0, last, s0sem)
    scatter_wait(hbuf0, last, s0sem)
    scatter_wait(hbuf1, last - 1, s1sem)

    plsc.subcore_barrier()
    for i in range((nrowch + NS - 1) // NS):
        c = i * NS + sid

        @pl.when(c < nrowch)
        def _():
            pltpu.sync_copy(out_sh.at[pl.ds(c * CH, CH)],
                            outp_hbm.at[cid, pl.ds(c * CH, CH)])


def kernel(x, edge_index, W, a):
    aa = a.reshape(2, D).T
    row3 = edge_index[0].reshape(NW, NCH, CH)
    col3 = edge_index[1].reshape(NW, NCH, CH)

    hb, s, cvec = pl.pallas_call(
        _prep_body,
        out_shape=[
            jax.ShapeDtypeStruct((N, D), jnp.bfloat16),
            jax.ShapeDtypeStruct((N, 2), jnp.float32),
            jax.ShapeDtypeStruct((LANES,), jnp.float32),
        ],
    )(x, W, aa)
    s1 = s[:, 0]
    s2 = s[:, 1]

    cp = _sc_compiler_params()
    mesh = plsc.VectorSubcoreMesh(core_axis_name="c", subcore_axis_name="s")

    edge_w_fn = pl.kernel(
        _edge_w_body,
        out_type=[
            jax.ShapeDtypeStruct((NW, EPW), jnp.float32),
            jax.ShapeDtypeStruct((NW, N), jnp.float32),
        ],
        mesh=mesh,
        scratch_types=[
            pltpu.VMEM((N,), jnp.float32),
            pltpu.VMEM((N,), jnp.float32),
            pltpu.VMEM((NCH, CH), jnp.int32),
            pltpu.VMEM((NCH, CH), jnp.int32),
            pltpu.VMEM((EPW,), jnp.float32),
            pltpu.VMEM((N,), jnp.float32),
            pltpu.VMEM((LANES,), jnp.float32),
        ],
        compiler_params=cp,
    )
    w3, ssum_p = edge_w_fn(row3, col3, s1, s2, cvec)

    spmm_fn = pl.kernel(
        _spmm_body,
        out_type=jax.ShapeDtypeStruct((NC, N, D), jnp.bfloat16),
        mesh=mesh,
        scratch_types=[
            pltpu.VMEM((NCH, CH), jnp.int32),
            pltpu.VMEM((NCH, CH), jnp.int32),
            pltpu.VMEM((EPW,), jnp.float32),
            pltpu.VMEM((CH, D), jnp.bfloat16),
            pltpu.VMEM((CH, D), jnp.bfloat16),
            pltpu.VMEM_SHARED((N, D), jnp.bfloat16),
            pltpu.SemaphoreType.DMA,
            pltpu.SemaphoreType.DMA,
            pltpu.SemaphoreType.DMA,
            pltpu.SemaphoreType.DMA,
        ],
        compiler_params=cp,
    )
    outp = spmm_fn(row3, col3, w3, hb)

    out = pl.pallas_call(
        _finish_body,
        out_shape=jax.ShapeDtypeStruct((N, D), jnp.float32),
    )(ssum_p, outp)
    return out


# speedup vs baseline: 34.4624x; 1.0216x over previous
import dataclasses

import jax
import jax.numpy as jnp
from jax import lax
from jax.experimental import pallas as pl
from jax.experimental.pallas import tpu as pltpu
from jax.experimental.pallas import tpu_sc as plsc

N = 10000
E = 320000
D = 128
ALPHA = 0.2
NC = 2
NS = 16
NW = NC * NS
EPW = E // NW
CH = 80
NCH = EPW // CH
NBUF = 4
LANES = 16


def _sc_compiler_params():
    cp = pltpu.CompilerParams()
    fields = pltpu.CompilerParams.__dataclass_fields__
    if "needs_layout_passes" in fields:
        cp = dataclasses.replace(cp, needs_layout_passes=False)
    if "use_tc_tiling_on_sc" in fields:
        cp = dataclasses.replace(cp, use_tc_tiling_on_sc=False)
    return cp


def _prep_body(x_ref, w_ref, aa_ref, hb_ref, s_ref, c_ref):
    h = jnp.dot(x_ref[...], w_ref[...], preferred_element_type=jnp.float32)
    hb_ref[...] = h.astype(jnp.bfloat16)
    s = jnp.dot(h, aa_ref[...], preferred_element_type=jnp.float32)
    s_ref[...] = s
    c = jnp.maximum(jnp.max(s[:, 0]) + jnp.max(s[:, 1]), 0.0)
    c_ref[...] = jnp.broadcast_to(c, (LANES,))


def _finish_body(ssum_ref, p_ref, o_ref):
    ssum = jnp.sum(ssum_ref[...], axis=0)
    sinv = 1.0 / jnp.maximum(ssum, 1e-8)
    p = p_ref[...].astype(jnp.float32)
    acc = (p[0] + p[1]) * sinv[:, None]
    o_ref[...] = jnp.where(acc > 0, acc, jnp.exp(acc) - 1.0)


def _edge_body(row_hbm, col_hbm, s1_hbm, s2_hbm, c_hbm, h_hbm,
               ssum_hbm, outp_hbm,
               s1_ts, s2_ts, row_ts, col_ts, w_ts, ssum_ts, c_ts,
               hbufs, out_sh, gsems, ssems):
    cid = lax.axis_index("c")
    sid = lax.axis_index("s")
    wid = sid * NC + cid

    pltpu.sync_copy(s1_hbm, s1_ts)
    pltpu.sync_copy(s2_hbm, s2_ts)
    pltpu.sync_copy(row_hbm.at[wid], row_ts)
    pltpu.sync_copy(col_hbm.at[wid], col_ts)
    pltpu.sync_copy(c_hbm, c_ts)
    cvec = c_ts[...]

    zf32 = jnp.zeros((LANES,), jnp.float32)

    @pl.loop(0, N, step=LANES)
    def _(i):
        ssum_ts[pl.ds(i, LANES)] = zf32

    @pl.loop(0, NCH)
    def _(j):
        @pl.loop(0, CH, step=LANES)
        def _(k):
            r = row_ts[j, pl.ds(k, LANES)]
            c = col_ts[j, pl.ds(k, LANES)]
            g1 = plsc.load_gather(s1_ts, [r])
            g2 = plsc.load_gather(s2_ts, [c])
            z = g1 + g2
            e = jnp.maximum(z, ALPHA * z)
            w = jnp.exp(e - cvec)
            w_ts[pl.ds(j * CH + k, LANES)] = w
            plsc.addupdate_scatter(ssum_ts, [r], w)

    pltpu.sync_copy(ssum_ts, ssum_hbm.at[wid])

    zbf = jnp.zeros((2 * LANES,), jnp.bfloat16)

    @pl.loop(0, CH)
    def _(i):
        for t in range(D // (2 * LANES)):
            hbufs[0][i, pl.ds(t * 2 * LANES, 2 * LANES)] = zbf

    nrowch = N // CH
    for i in range((nrowch + NS - 1) // NS):
        c = i * NS + sid

        @pl.when(c < nrowch)
        def _():
            pltpu.sync_copy(hbufs[0], out_sh.at[pl.ds(c * CH, CH)])

    plsc.subcore_barrier()

    def scale(buf, j):
        @pl.loop(0, CH, step=LANES)
        def _(kk):
            base = j * CH + kk
            for u in range(LANES):
                wv = plsc.load_gather(
                    w_ts, [jnp.full((LANES,), base + u, jnp.int32)])
                wb = plsc.pack(wv, wv, format=plsc.PackFormat.INTERLEAVED)
                for t in range(D // (2 * LANES)):
                    sl = pl.ds(t * 2 * LANES, 2 * LANES)
                    buf[kk + u, sl] = buf[kk + u, sl] * wb

    def gather_start(b, j):
        pltpu.async_copy(h_hbm.at[col_ts.at[j]], hbufs[b], gsems[b])

    def gather_wait(b, j):
        pltpu.make_async_copy(h_hbm.at[col_ts.at[j]], hbufs[b],
                              gsems[b]).wait()

    def scatter_start(b, j):
        pltpu.async_copy(hbufs[b], out_sh.at[row_ts.at[j]], ssems[b],
                         add=True)

    def scatter_wait(b, j):
        pltpu.make_async_copy(hbufs[b], out_sh.at[row_ts.at[j]],
                              ssems[b]).wait()

    for b in range(NBUF - 1):
        gather_start(b, b)

    def step(j, b):
        bprev = (b - 1) % NBUF

        @pl.when(j >= 1)
        def _():
            scatter_wait(bprev, j - 1)

        @pl.when(j + NBUF - 1 < NCH)
        def _():
            gather_start(bprev, j + NBUF - 1)

        gather_wait(b, j)
        scale(hbufs[b], j)
        scatter_start(b, j)

    @pl.loop(0, NCH // NBUF)
    def _(m):
        for b in range(NBUF):
            step(m * NBUF + b, b)

    for r in range((NCH // NBUF) * NBUF, NCH):
        step(r, r % NBUF)

    scatter_wait((NCH - 1) % NBUF, NCH - 1)

    plsc.subcore_barrier()
    for i in range((nrowch + NS - 1) // NS):
        c = i * NS + sid

        @pl.when(c < nrowch)
        def _():
            pltpu.sync_copy(out_sh.at[pl.ds(c * CH, CH)],
                            outp_hbm.at[cid, pl.ds(c * CH, CH)])


def kernel(x, edge_index, W, a):
    aa = a.reshape(2, D).T
    row3 = edge_index[0].reshape(NW, NCH, CH)
    col3 = edge_index[1].reshape(NW, NCH, CH)

    hb, s, cvec = pl.pallas_call(
        _prep_body,
        out_shape=[
            jax.ShapeDtypeStruct((N, D), jnp.bfloat16),
            jax.ShapeDtypeStruct((N, 2), jnp.float32),
            jax.ShapeDtypeStruct((LANES,), jnp.float32),
        ],
    )(x, W, aa)
    s1 = s[:, 0]
    s2 = s[:, 1]

    cp = _sc_compiler_params()
    mesh = plsc.VectorSubcoreMesh(core_axis_name="c", subcore_axis_name="s")

    edge_fn = pl.kernel(
        _edge_body,
        out_type=[
            jax.ShapeDtypeStruct((NW, N), jnp.float32),
            jax.ShapeDtypeStruct((NC, N, D), jnp.bfloat16),
        ],
        mesh=mesh,
        scratch_types=[
            pltpu.VMEM((N,), jnp.float32),
            pltpu.VMEM((N,), jnp.float32),
            pltpu.VMEM((NCH, CH), jnp.int32),
            pltpu.VMEM((NCH, CH), jnp.int32),
            pltpu.VMEM((EPW,), jnp.float32),
            pltpu.VMEM((N,), jnp.float32),
            pltpu.VMEM((LANES,), jnp.float32),
            [pltpu.VMEM((CH, D), jnp.bfloat16) for _ in range(NBUF)],
            pltpu.VMEM_SHARED((N, D), jnp.bfloat16),
            [pltpu.SemaphoreType.DMA for _ in range(NBUF)],
            [pltpu.SemaphoreType.DMA for _ in range(NBUF)],
        ],
        compiler_params=cp,
    )
    ssum_p, outp = edge_fn(row3, col3, s1, s2, cvec, hb)

    out = pl.pallas_call(
        _finish_body,
        out_shape=jax.ShapeDtypeStruct((N, D), jnp.float32),
    )(ssum_p, outp)
    return out


# speedup vs baseline: 36.2868x; 1.0529x over previous
import dataclasses

import jax
import jax.numpy as jnp
from jax import lax
from jax.experimental import pallas as pl
from jax.experimental.pallas import tpu as pltpu
from jax.experimental.pallas import tpu_sc as plsc

N = 10000
E = 320000
D = 128
ALPHA = 0.2
NC = 2
NS = 16
NW = NC * NS
EPW = E // NW
CH = 80
NCH = EPW // CH
NBUF = 4
LANES = 16


def _sc_compiler_params():
    cp = pltpu.CompilerParams()
    fields = pltpu.CompilerParams.__dataclass_fields__
    if "needs_layout_passes" in fields:
        cp = dataclasses.replace(cp, needs_layout_passes=False)
    if "use_tc_tiling_on_sc" in fields:
        cp = dataclasses.replace(cp, use_tc_tiling_on_sc=False)
    return cp


def _prep_body(x_ref, w_ref, a_ref, hb_ref, s_ref, c_ref):
    h = jnp.dot(x_ref[...], w_ref[...], preferred_element_type=jnp.float32)
    hb_ref[...] = h.astype(jnp.bfloat16)
    s1 = jnp.dot(h, a_ref[:D, :], preferred_element_type=jnp.float32)
    s2 = jnp.dot(h, a_ref[D:, :], preferred_element_type=jnp.float32)
    s_ref[0, :] = s1[:, 0]
    s_ref[1, :] = s2[:, 0]
    c = jnp.maximum(jnp.max(s1) + jnp.max(s2), 0.0)
    c_ref[...] = jnp.broadcast_to(c, (LANES,))


def _finish_body(ssum_ref, p_ref, o_ref):
    ssum = jnp.sum(ssum_ref[...], axis=0)
    sinv = 1.0 / jnp.maximum(ssum, 1e-8)
    p = p_ref[...].astype(jnp.float32)
    acc = (p[0] + p[1]) * sinv[:, None]
    o_ref[...] = jnp.where(acc > 0, acc, jnp.exp(acc) - 1.0)


def _edge_body(ei_hbm, s_hbm, c_hbm, h_hbm,
               ssum_hbm, outp_hbm,
               s1_ts, s2_ts, row_ts, col_ts, w_ts, ssum_ts, c_ts,
               hbufs, out_sh, gsems, ssems):
    cid = lax.axis_index("c")
    sid = lax.axis_index("s")
    wid = sid * NC + cid

    pltpu.sync_copy(s_hbm.at[0], s1_ts)
    pltpu.sync_copy(s_hbm.at[1], s2_ts)
    pltpu.sync_copy(ei_hbm.at[0, wid], row_ts)
    pltpu.sync_copy(ei_hbm.at[1, wid], col_ts)
    pltpu.sync_copy(c_hbm, c_ts)
    cvec = c_ts[...]

    zf32 = jnp.zeros((LANES,), jnp.float32)

    @pl.loop(0, N, step=LANES)
    def _(i):
        ssum_ts[pl.ds(i, LANES)] = zf32

    @pl.loop(0, NCH)
    def _(j):
        @pl.loop(0, CH, step=LANES)
        def _(k):
            r = row_ts[j, pl.ds(k, LANES)]
            c = col_ts[j, pl.ds(k, LANES)]
            g1 = plsc.load_gather(s1_ts, [r])
            g2 = plsc.load_gather(s2_ts, [c])
            z = g1 + g2
            e = jnp.maximum(z, ALPHA * z)
            w = jnp.exp(e - cvec)
            w_ts[pl.ds(j * CH + k, LANES)] = w
            plsc.addupdate_scatter(ssum_ts, [r], w)

    pltpu.sync_copy(ssum_ts, ssum_hbm.at[wid])

    zbf = jnp.zeros((2 * LANES,), jnp.bfloat16)

    @pl.loop(0, CH)
    def _(i):
        for t in range(D // (2 * LANES)):
            hbufs[0][i, pl.ds(t * 2 * LANES, 2 * LANES)] = zbf

    nrowch = N // CH
    for i in range((nrowch + NS - 1) // NS):
        c = i * NS + sid

        @pl.when(c < nrowch)
        def _():
            pltpu.sync_copy(hbufs[0], out_sh.at[pl.ds(c * CH, CH)])

    plsc.subcore_barrier()

    def scale(buf, j):
        @pl.loop(0, CH, step=LANES)
        def _(kk):
            base = j * CH + kk
            for u in range(LANES):
                wv = plsc.load_gather(
                    w_ts, [jnp.full((LANES,), base + u, jnp.int32)])
                wb = plsc.pack(wv, wv, format=plsc.PackFormat.INTERLEAVED)
                for t in range(D // (2 * LANES)):
                    sl = pl.ds(t * 2 * LANES, 2 * LANES)
                    buf[kk + u, sl] = buf[kk + u, sl] * wb

    def gather_start(b, j):
        pltpu.async_copy(h_hbm.at[col_ts.at[j]], hbufs[b], gsems[b])

    def gather_wait(b, j):
        pltpu.make_async_copy(h_hbm.at[col_ts.at[j]], hbufs[b],
                              gsems[b]).wait()

    def scatter_start(b, j):
        pltpu.async_copy(hbufs[b], out_sh.at[row_ts.at[j]], ssems[b],
                         add=True)

    def scatter_wait(b, j):
        pltpu.make_async_copy(hbufs[b], out_sh.at[row_ts.at[j]],
                              ssems[b]).wait()

    for b in range(NBUF - 1):
        gather_start(b, b)

    def step(j, b):
        bprev = (b - 1) % NBUF

        @pl.when(j >= 1)
        def _():
            scatter_wait(bprev, j - 1)

        @pl.when(j + NBUF - 1 < NCH)
        def _():
            gather_start(bprev, j + NBUF - 1)

        gather_wait(b, j)
        scale(hbufs[b], j)
        scatter_start(b, j)

    @pl.loop(0, NCH // NBUF)
    def _(m):
        for b in range(NBUF):
            step(m * NBUF + b, b)

    for r in range((NCH // NBUF) * NBUF, NCH):
        step(r, r % NBUF)

    scatter_wait((NCH - 1) % NBUF, NCH - 1)

    plsc.subcore_barrier()
    for i in range((nrowch + NS - 1) // NS):
        c = i * NS + sid

        @pl.when(c < nrowch)
        def _():
            pltpu.sync_copy(out_sh.at[pl.ds(c * CH, CH)],
                            outp_hbm.at[cid, pl.ds(c * CH, CH)])


def kernel(x, edge_index, W, a):
    ei4 = edge_index.reshape(2, NW, NCH, CH)

    hb, s, cvec = pl.pallas_call(
        _prep_body,
        out_shape=[
            jax.ShapeDtypeStruct((N, D), jnp.bfloat16),
            jax.ShapeDtypeStruct((2, N), jnp.float32),
            jax.ShapeDtypeStruct((LANES,), jnp.float32),
        ],
    )(x, W, a)

    cp = _sc_compiler_params()
    mesh = plsc.VectorSubcoreMesh(core_axis_name="c", subcore_axis_name="s")

    edge_fn = pl.kernel(
        _edge_body,
        out_type=[
            jax.ShapeDtypeStruct((NW, N), jnp.float32),
            jax.ShapeDtypeStruct((NC, N, D), jnp.bfloat16),
        ],
        mesh=mesh,
        scratch_types=[
            pltpu.VMEM((N,), jnp.float32),
            pltpu.VMEM((N,), jnp.float32),
            pltpu.VMEM((NCH, CH), jnp.int32),
            pltpu.VMEM((NCH, CH), jnp.int32),
            pltpu.VMEM((EPW,), jnp.float32),
            pltpu.VMEM((N,), jnp.float32),
            pltpu.VMEM((LANES,), jnp.float32),
            [pltpu.VMEM((CH, D), jnp.bfloat16) for _ in range(NBUF)],
            pltpu.VMEM_SHARED((N, D), jnp.bfloat16),
            [pltpu.SemaphoreType.DMA for _ in range(NBUF)],
            [pltpu.SemaphoreType.DMA for _ in range(NBUF)],
        ],
        compiler_params=cp,
    )
    ssum_p, outp = edge_fn(ei4, s, cvec, hb)

    out = pl.pallas_call(
        _finish_body,
        out_shape=jax.ShapeDtypeStruct((N, D), jnp.float32),
    )(ssum_p, outp)
    return out


# speedup vs baseline: 37.1945x; 1.0250x over previous
import dataclasses

import jax
import jax.numpy as jnp
from jax import lax
from jax.experimental import pallas as pl
from jax.experimental.pallas import tpu as pltpu
from jax.experimental.pallas import tpu_sc as plsc

N = 10000
E = 320000
D = 128
ALPHA = 0.2
NC = 2
NS = 16
NW = NC * NS
EPW = E // NW
CH = 80
NCH = EPW // CH
NBUF = 4
LANES = 16
NPAD = 10240


def _sc_compiler_params():
    cp = pltpu.CompilerParams()
    fields = pltpu.CompilerParams.__dataclass_fields__
    if "needs_layout_passes" in fields:
        cp = dataclasses.replace(cp, needs_layout_passes=False)
    if "use_tc_tiling_on_sc" in fields:
        cp = dataclasses.replace(cp, use_tc_tiling_on_sc=False)
    return cp


def _prep_body(x_ref, w_ref, a_ref, hb_ref, s_ref):
    h = jnp.dot(x_ref[...], w_ref[...], preferred_element_type=jnp.float32)
    hb_ref[...] = h.astype(jnp.bfloat16)
    s1 = jnp.dot(h, a_ref[:D, :], preferred_element_type=jnp.float32)
    s2 = jnp.dot(h, a_ref[D:, :], preferred_element_type=jnp.float32)
    c = jnp.maximum(jnp.max(s1) + jnp.max(s2), 0.0)
    s_ref[...] = jnp.broadcast_to(c, (8, NPAD))
    s_ref[0, :N] = s1[:, 0]
    s_ref[1, :N] = s2[:, 0]


def _finish_body(ssum_ref, p_ref, o_ref):
    ssum = jnp.sum(ssum_ref[...], axis=0)[:N]
    sinv = 1.0 / jnp.maximum(ssum, 1e-8)
    p = p_ref[...].astype(jnp.float32)
    acc = (p[0] + p[1]) * sinv[:, None]
    o_ref[...] = jnp.where(acc > 0, acc, jnp.exp(acc) - 1.0)


def _edge_body(ei_hbm, s_hbm, h_hbm,
               ssum_hbm, outp_hbm,
               s1_ts, s2_ts, row_ts, col_ts, w_ts, ssum_ts, c_ts,
               hbufs, out_sh, gsems, ssems):
    cid = lax.axis_index("c")
    sid = lax.axis_index("s")
    wid = sid * NC + cid

    pltpu.sync_copy(s_hbm.at[0, pl.ds(0, N)], s1_ts)
    pltpu.sync_copy(s_hbm.at[1, pl.ds(0, N)], s2_ts)
    pltpu.sync_copy(ei_hbm.at[0, pl.ds(wid * EPW, EPW)], row_ts)
    pltpu.sync_copy(ei_hbm.at[1, pl.ds(wid * EPW, EPW)], col_ts)
    pltpu.sync_copy(s_hbm.at[2, pl.ds(0, LANES)], c_ts)
    cvec = c_ts[...]

    zf32 = jnp.zeros((LANES,), jnp.float32)

    @pl.loop(0, NPAD, step=LANES)
    def _(i):
        ssum_ts[pl.ds(i, LANES)] = zf32

    @pl.loop(0, EPW, step=LANES)
    def _(k):
        r = row_ts[pl.ds(k, LANES)]
        c = col_ts[pl.ds(k, LANES)]
        g1 = plsc.load_gather(s1_ts, [r])
        g2 = plsc.load_gather(s2_ts, [c])
        z = g1 + g2
        e = jnp.maximum(z, ALPHA * z)
        w = jnp.exp(e - cvec)
        w_ts[pl.ds(k, LANES)] = w
        plsc.addupdate_scatter(ssum_ts, [r], w)

    pltpu.sync_copy(ssum_ts, ssum_hbm.at[wid])

    zbf = jnp.zeros((2 * LANES,), jnp.bfloat16)

    @pl.loop(0, CH)
    def _(i):
        for t in range(D // (2 * LANES)):
            hbufs[0][i, pl.ds(t * 2 * LANES, 2 * LANES)] = zbf

    nrowch = N // CH
    for i in range((nrowch + NS - 1) // NS):
        c = i * NS + sid

        @pl.when(c < nrowch)
        def _():
            pltpu.sync_copy(hbufs[0], out_sh.at[pl.ds(c * CH, CH)])

    plsc.subcore_barrier()

    def scale(buf, j):
        @pl.loop(0, CH, step=LANES)
        def _(kk):
            base = j * CH + kk
            for u in range(LANES):
                wv = plsc.load_gather(
                    w_ts, [jnp.full((LANES,), base + u, jnp.int32)])
                wb = plsc.pack(wv, wv, format=plsc.PackFormat.INTERLEAVED)
                for t in range(D // (2 * LANES)):
                    sl = pl.ds(t * 2 * LANES, 2 * LANES)
                    buf[kk + u, sl] = buf[kk + u, sl] * wb

    def gather_start(b, j):
        pltpu.async_copy(h_hbm.at[col_ts.at[pl.ds(j * CH, CH)]], hbufs[b],
                         gsems[b])

    def gather_wait(b, j):
        pltpu.make_async_copy(h_hbm.at[col_ts.at[pl.ds(j * CH, CH)]],
                              hbufs[b], gsems[b]).wait()

    def scatter_start(b, j):
        pltpu.async_copy(hbufs[b], out_sh.at[row_ts.at[pl.ds(j * CH, CH)]],
                         ssems[b], add=True)

    def scatter_wait(b, j):
        pltpu.make_async_copy(hbufs[b], out_sh.at[row_ts.at[pl.ds(j * CH, CH)]],
                              ssems[b]).wait()

    for b in range(NBUF - 1):
        gather_start(b, b)

    def step(j, b):
        bprev = (b - 1) % NBUF

        @pl.when(j >= 1)
        def _():
            scatter_wait(bprev, j - 1)

        @pl.when(j + NBUF - 1 < NCH)
        def _():
            gather_start(bprev, j + NBUF - 1)

        gather_wait(b, j)
        scale(hbufs[b], j)
        scatter_start(b, j)

    @pl.loop(0, NCH // NBUF)
    def _(m):
        for b in range(NBUF):
            step(m * NBUF + b, b)

    for r in range((NCH // NBUF) * NBUF, NCH):
        step(r, r % NBUF)

    scatter_wait((NCH - 1) % NBUF, NCH - 1)

    plsc.subcore_barrier()
    for i in range((nrowch + NS - 1) // NS):
        c = i * NS + sid

        @pl.when(c < nrowch)
        def _():
            pltpu.sync_copy(out_sh.at[pl.ds(c * CH, CH)],
                            outp_hbm.at[cid, pl.ds(c * CH, CH)])


def kernel(x, edge_index, W, a):
    hb, s = pl.pallas_call(
        _prep_body,
        out_shape=[
            jax.ShapeDtypeStruct((N, D), jnp.bfloat16),
            jax.ShapeDtypeStruct((8, NPAD), jnp.float32),
        ],
    )(x, W, a)

    cp = _sc_compiler_params()
    mesh = plsc.VectorSubcoreMesh(core_axis_name="c", subcore_axis_name="s")

    edge_fn = pl.kernel(
        _edge_body,
        out_type=[
            jax.ShapeDtypeStruct((NW, NPAD), jnp.float32),
            jax.ShapeDtypeStruct((NC, N, D), jnp.bfloat16),
        ],
        mesh=mesh,
        scratch_types=[
            pltpu.VMEM((N,), jnp.float32),
            pltpu.VMEM((N,), jnp.float32),
            pltpu.VMEM((EPW,), jnp.int32),
            pltpu.VMEM((EPW,), jnp.int32),
            pltpu.VMEM((EPW,), jnp.float32),
            pltpu.VMEM((NPAD,), jnp.float32),
            pltpu.VMEM((LANES,), jnp.float32),
            [pltpu.VMEM((CH, D), jnp.bfloat16) for _ in range(NBUF)],
            pltpu.VMEM_SHARED((N, D), jnp.bfloat16),
            [pltpu.SemaphoreType.DMA for _ in range(NBUF)],
            [pltpu.SemaphoreType.DMA for _ in range(NBUF)],
        ],
        compiler_params=cp,
    )
    ssum_p, outp = edge_fn(edge_index, s, hb)

    out = pl.pallas_call(
        _finish_body,
        out_shape=jax.ShapeDtypeStruct((N, D), jnp.float32),
    )(ssum_p, outp)
    return out


# speedup vs baseline: 37.6192x; 1.0114x over previous
import dataclasses

import jax
import jax.numpy as jnp
from jax import lax
from jax.experimental import pallas as pl
from jax.experimental.pallas import tpu as pltpu
from jax.experimental.pallas import tpu_sc as plsc

N = 10000
E = 320000
D = 128
ALPHA = 0.2
NC = 2
NS = 16
NW = NC * NS
EPW = E // NW
CH = 80
NCH = EPW // CH
NBUF = 4
LANES = 16
NPAD = 10240


def _sc_compiler_params():
    cp = pltpu.CompilerParams()
    fields = pltpu.CompilerParams.__dataclass_fields__
    if "needs_layout_passes" in fields:
        cp = dataclasses.replace(cp, needs_layout_passes=False)
    if "use_tc_tiling_on_sc" in fields:
        cp = dataclasses.replace(cp, use_tc_tiling_on_sc=False)
    return cp


def _prep_body(x_ref, w_ref, a_ref, hb_ref, s_ref):
    h = jnp.dot(x_ref[...], w_ref[...], preferred_element_type=jnp.float32)
    hb_ref[...] = h.astype(jnp.bfloat16)
    s1 = jnp.dot(h, a_ref[:D, :], preferred_element_type=jnp.float32)
    s2 = jnp.dot(h, a_ref[D:, :], preferred_element_type=jnp.float32)
    c = jnp.maximum(jnp.max(s1) + jnp.max(s2), 0.0)
    zpad = jnp.zeros((NPAD - N,), jnp.float32)
    s_ref[0, :] = jnp.concatenate([s1[:, 0], zpad])
    s_ref[1, :] = jnp.concatenate([s2[:, 0], zpad])
    s_ref[2, :] = jnp.broadcast_to(c, (NPAD,))


def _finish_body(ssum_ref, p_ref, o_ref):
    ssum = jnp.sum(ssum_ref[...], axis=0)[:N]
    sinv = 1.0 / jnp.maximum(ssum, 1e-8)
    p = p_ref[...].astype(jnp.float32)
    acc = (p[0] + p[1]) * sinv[:, None]
    o_ref[...] = jnp.where(acc > 0, acc, jnp.exp(acc) - 1.0)


def _edge_body(ei_hbm, s_hbm, h_hbm,
               ssum_hbm, outp_hbm,
               s1_ts, s2_ts, row_ts, col_ts, w_ts, ssum_ts, c_ts,
               hbufs, out_sh, gsems, ssems):
    cid = lax.axis_index("c")
    sid = lax.axis_index("s")
    wid = sid * NC + cid

    d1 = pltpu.async_copy(s_hbm.at[0, pl.ds(0, N)], s1_ts, gsems[0])
    d2 = pltpu.async_copy(s_hbm.at[1, pl.ds(0, N)], s2_ts, gsems[1])
    d3 = pltpu.async_copy(ei_hbm.at[0, pl.ds(wid * EPW, EPW)], row_ts,
                          gsems[2])
    d4 = pltpu.async_copy(ei_hbm.at[1, pl.ds(wid * EPW, EPW)], col_ts,
                          gsems[3])
    d5 = pltpu.async_copy(s_hbm.at[2, pl.ds(0, LANES)], c_ts, ssems[0])
    d1.wait(); d2.wait(); d3.wait(); d4.wait(); d5.wait()
    cvec = c_ts[...]

    zf32 = jnp.zeros((LANES,), jnp.float32)

    @pl.loop(0, NPAD, step=LANES)
    def _(i):
        ssum_ts[pl.ds(i, LANES)] = zf32

    @pl.loop(0, EPW, step=LANES)
    def _(k):
        r = row_ts[pl.ds(k, LANES)]
        c = col_ts[pl.ds(k, LANES)]
        g1 = plsc.load_gather(s1_ts, [r])
        g2 = plsc.load_gather(s2_ts, [c])
        z = g1 + g2
        e = jnp.maximum(z, ALPHA * z)
        w = jnp.exp(e - cvec)
        w_ts[pl.ds(k, LANES)] = w
        plsc.addupdate_scatter(ssum_ts, [r], w)

    pltpu.sync_copy(ssum_ts, ssum_hbm.at[wid])

    zbf = jnp.zeros((2 * LANES,), jnp.bfloat16)

    @pl.loop(0, CH)
    def _(i):
        for t in range(D // (2 * LANES)):
            hbufs[0][i, pl.ds(t * 2 * LANES, 2 * LANES)] = zbf

    nrowch = N // CH
    for i in range((nrowch + NS - 1) // NS):
        c = i * NS + sid

        @pl.when(c < nrowch)
        def _():
            pltpu.sync_copy(hbufs[0], out_sh.at[pl.ds(c * CH, CH)])

    plsc.subcore_barrier()

    def scale(buf, j):
        @pl.loop(0, CH, step=LANES)
        def _(kk):
            base = j * CH + kk
            for u in range(LANES):
                wv = plsc.load_gather(
                    w_ts, [jnp.full((LANES,), base + u, jnp.int32)])
                wb = plsc.pack(wv, wv, format=plsc.PackFormat.INTERLEAVED)
                for t in range(D // (2 * LANES)):
                    sl = pl.ds(t * 2 * LANES, 2 * LANES)
                    buf[kk + u, sl] = buf[kk + u, sl] * wb

    def gather_start(b, j):
        pltpu.async_copy(h_hbm.at[col_ts.at[pl.ds(j * CH, CH)]], hbufs[b],
                         gsems[b])

    def gather_wait(b, j):
        pltpu.make_async_copy(h_hbm.at[col_ts.at[pl.ds(j * CH, CH)]],
                              hbufs[b], gsems[b]).wait()

    def scatter_start(b, j):
        pltpu.async_copy(hbufs[b], out_sh.at[row_ts.at[pl.ds(j * CH, CH)]],
                         ssems[b], add=True)

    def scatter_wait(b, j):
        pltpu.make_async_copy(hbufs[b], out_sh.at[row_ts.at[pl.ds(j * CH, CH)]],
                              ssems[b]).wait()

    for b in range(NBUF - 1):
        gather_start(b, b)

    def step(j, b):
        bprev = (b - 1) % NBUF

        @pl.when(j >= 1)
        def _():
            scatter_wait(bprev, j - 1)

        @pl.when(j + NBUF - 1 < NCH)
        def _():
            gather_start(bprev, j + NBUF - 1)

        gather_wait(b, j)
        scale(hbufs[b], j)
        scatter_start(b, j)

    @pl.loop(0, NCH // NBUF)
    def _(m):
        for b in range(NBUF):
            step(m * NBUF + b, b)

    for r in range((NCH // NBUF) * NBUF, NCH):
        step(r, r % NBUF)

    scatter_wait((NCH - 1) % NBUF, NCH - 1)

    plsc.subcore_barrier()
    for i in range((nrowch + NS - 1) // NS):
        c = i * NS + sid

        @pl.when(c < nrowch)
        def _():
            pltpu.sync_copy(out_sh.at[pl.ds(c * CH, CH)],
                            outp_hbm.at[cid, pl.ds(c * CH, CH)])


def kernel(x, edge_index, W, a):
    hb, s = pl.pallas_call(
        _prep_body,
        out_shape=[
            jax.ShapeDtypeStruct((N, D), jnp.bfloat16),
            jax.ShapeDtypeStruct((8, NPAD), jnp.float32),
        ],
    )(x, W, a)

    cp = _sc_compiler_params()
    mesh = plsc.VectorSubcoreMesh(core_axis_name="c", subcore_axis_name="s")

    edge_fn = pl.kernel(
        _edge_body,
        out_type=[
            jax.ShapeDtypeStruct((NW, NPAD), jnp.float32),
            jax.ShapeDtypeStruct((NC, N, D), jnp.bfloat16),
        ],
        mesh=mesh,
        scratch_types=[
            pltpu.VMEM((N,), jnp.float32),
            pltpu.VMEM((N,), jnp.float32),
            pltpu.VMEM((EPW,), jnp.int32),
            pltpu.VMEM((EPW,), jnp.int32),
            pltpu.VMEM((EPW,), jnp.float32),
            pltpu.VMEM((NPAD,), jnp.float32),
            pltpu.VMEM((LANES,), jnp.float32),
            [pltpu.VMEM((CH, D), jnp.bfloat16) for _ in range(NBUF)],
            pltpu.VMEM_SHARED((N, D), jnp.bfloat16),
            [pltpu.SemaphoreType.DMA for _ in range(NBUF)],
            [pltpu.SemaphoreType.DMA for _ in range(NBUF)],
        ],
        compiler_params=cp,
    )
    ssum_p, outp = edge_fn(edge_index, s, hb)

    out = pl.pallas_call(
        _finish_body,
        out_shape=jax.ShapeDtypeStruct((N, D), jnp.float32),
    )(ssum_p, outp)
    return out
